# baseline (speedup 1.0000x reference)
import functools

import jax
import jax.numpy as jnp
from jax import lax
from jax.experimental import pallas as pl
from jax.experimental.pallas import tpu as pltpu
from jax.experimental.pallas import tpu_sc as plsc

D = 128


def _make_sc_gather(B):
  info = plsc.get_sparse_core_info()
  NC, NS = info.num_cores, info.num_subcores
  NW = NC * NS
  assert B % (8 * NW) == 0
  b_per_w = B // NW
  CH = 128
  n_ch = b_per_w // CH
  NBUF = 4
  n_jobs = 2 * n_ch
  mesh = plsc.VectorSubcoreMesh(core_axis_name="c", subcore_axis_name="s")

  out_type = (
      jax.ShapeDtypeStruct((B, D), jnp.float32),
      jax.ShapeDtypeStruct((B, D), jnp.float32),
  )

  @functools.partial(
      pl.kernel,
      mesh=mesh,
      out_type=out_type,
      scratch_types=(
          [pltpu.VMEM((b_per_w,), jnp.int32) for _ in range(2)]
          + [pltpu.VMEM((CH, D), jnp.float32) for _ in range(NBUF)]
          + [pltpu.SemaphoreType.DMA for _ in range(2 * NBUF)]
      ),
  )
  def gather(uidx, iidx, ut, it, uout, iout,
             uidx_v, iidx_v, b0, b1, b2, b3, *sems):
    bufs = (b0, b1, b2, b3)
    gsem = sems[:NBUF]
    ssem = sems[NBUF:]
    wid = lax.axis_index("s") * NC + lax.axis_index("c")
    base = wid * b_per_w
    pltpu.sync_copy(uidx.at[pl.ds(base, b_per_w)], uidx_v)
    pltpu.sync_copy(iidx.at[pl.ds(base, b_per_w)], iidx_v)

    def job(j):
      if j < n_ch:
        return uidx_v, ut, uout, j * CH
      return iidx_v, it, iout, (j - n_ch) * CH

    gd = [None] * NBUF
    sd = [None] * NBUF
    for k in range(NBUF):
      idx_v, tab, _, off = job(k)
      gd[k] = pltpu.make_async_copy(tab.at[idx_v.at[pl.ds(off, CH)]],
                                    bufs[k], gsem[k])
      gd[k].start()
    for j in range(n_jobs):
      b = j % NBUF
      _, _, out, off = job(j)
      gd[b].wait()
      sd[b] = pltpu.make_async_copy(bufs[b], out.at[pl.ds(base + off, CH)],
                                    ssem[b])
      sd[b].start()
      nj = j + NBUF
      if nj < n_jobs:
        sd[b].wait()
        idx_v, tab, _, noff = job(nj)
        gd[b] = pltpu.make_async_copy(tab.at[idx_v.at[pl.ds(noff, CH)]],
                                      bufs[b], gsem[b])
        gd[b].start()
    for b in range(NBUF):
      sd[b].wait()

  return gather


def _mlp_body(u_ref, i_ref, a_ref, g_ref, c_ref, p_ref,
              at_ref, gt_ref, ct_ref,
              w1_ref, b1r,
              w2, b2r, w3, b3r, w4, b4r, o_ref):
  f32 = jnp.float32
  T = u_ref.shape[0]

  def onehot_emb(idx_ref, tab_ref):
    n = tab_ref.shape[0]
    iot = lax.broadcasted_iota(jnp.int32, (T, n), 1)
    oh = (iot == idx_ref[...]).astype(f32)
    return jnp.dot(oh, tab_ref[...], preferred_element_type=f32)

  w1 = w1_ref[...]
  h = jnp.dot(u_ref[...], w1[0:128], preferred_element_type=f32)
  h = h + jnp.dot(i_ref[...], w1[128:256], preferred_element_type=f32)
  h = h + jnp.dot(onehot_emb(a_ref, at_ref), w1[256:320],
                  preferred_element_type=f32)
  h = h + jnp.dot(onehot_emb(g_ref, gt_ref), w1[320:352],
                  preferred_element_type=f32)
  h = h + jnp.dot(onehot_emb(c_ref, ct_ref), w1[352:416],
                  preferred_element_type=f32)
  h = h + (p_ref[...] * (1.0 / 1000.0)) * w1[416:417]
  h = jnp.maximum(h + b1r[...], 0.0)
  h = jnp.maximum(jnp.dot(h, w2[...], preferred_element_type=f32) + b2r[...],
                  0.0)
  h = jnp.maximum(jnp.dot(h, w3[...], preferred_element_type=f32) + b3r[...],
                  0.0)
  o_ref[...] = jnp.dot(h, w4[...], preferred_element_type=f32) + b4r[...]


def kernel(user_id, age, gender, item_id, category, price,
           user_table, item_table, age_table, gender_table, cat_table,
           W1, b1, W2, b2, W3, b3, W4, b4):
  B = user_id.shape[0]

  uidx = user_id.reshape(B).astype(jnp.int32)
  iidx = item_id.reshape(B).astype(jnp.int32)

  u_emb, i_emb = _make_sc_gather(B)(uidx, iidx, user_table, item_table)

  TILE = 4096
  grid = (B // TILE,)

  def tile_spec(w):
    return pl.BlockSpec((TILE, w), lambda i: (i, 0))

  def full_spec(a):
    r, c = a.shape
    return pl.BlockSpec((r, c), lambda i: (0, 0))

  b1r = b1.reshape(1, -1)
  b2r = b2.reshape(1, -1)
  b3r = b3.reshape(1, -1)
  b4r = b4.reshape(1, -1)

  out = pl.pallas_call(
      _mlp_body,
      grid=grid,
      in_specs=[
          tile_spec(D), tile_spec(D),
          tile_spec(1), tile_spec(1), tile_spec(1), tile_spec(1),
          full_spec(age_table), full_spec(gender_table), full_spec(cat_table),
          full_spec(W1), full_spec(b1r),
          full_spec(W2), full_spec(b2r),
          full_spec(W3), full_spec(b3r),
          full_spec(W4), full_spec(b4r),
      ],
      out_specs=pl.BlockSpec((TILE, 1), lambda i: (i, 0)),
      out_shape=jax.ShapeDtypeStruct((B, 1), jnp.float32),
      compiler_params=pltpu.CompilerParams(
          dimension_semantics=("parallel",)),
  )(u_emb, i_emb, age, gender, category, price,
    age_table, gender_table, cat_table,
    W1, b1r, W2, b2r, W3, b3r, W4, b4r)

  return out[:, 0]


# speedup vs baseline: 1.0092x; 1.0092x over previous
import functools

import jax
import jax.numpy as jnp
from jax import lax
from jax.experimental import pallas as pl
from jax.experimental.pallas import tpu as pltpu
from jax.experimental.pallas import tpu_sc as plsc

D = 128


def _make_sc_gather(B):
  info = plsc.get_sparse_core_info()
  NC, NS = info.num_cores, info.num_subcores
  NW = NC * NS
  assert B % (8 * NW) == 0
  b_per_w = B // NW
  CH = 128
  n_ch = b_per_w // CH
  NBUF = 4
  n_jobs = 2 * n_ch
  mesh = plsc.VectorSubcoreMesh(core_axis_name="c", subcore_axis_name="s")

  out_type = (
      jax.ShapeDtypeStruct((B, D), jnp.float32),
      jax.ShapeDtypeStruct((B, D), jnp.float32),
  )

  @functools.partial(
      pl.kernel,
      mesh=mesh,
      out_type=out_type,
      scratch_types=(
          [pltpu.VMEM((b_per_w,), jnp.int32) for _ in range(2)]
          + [pltpu.VMEM((CH, D), jnp.float32) for _ in range(NBUF)]
          + [pltpu.SemaphoreType.DMA for _ in range(2 * NBUF)]
      ),
      compiler_params=pltpu.CompilerParams(use_tc_tiling_on_sc=True),
  )
  def gather(uidx, iidx, ut, it, uout, iout,
             uidx_v, iidx_v, b0, b1, b2, b3, *sems):
    bufs = (b0, b1, b2, b3)
    gsem = sems[:NBUF]
    ssem = sems[NBUF:]
    wid = lax.axis_index("s") * NC + lax.axis_index("c")
    base = wid * b_per_w
    pltpu.sync_copy(uidx.at[pl.ds(base, b_per_w)], uidx_v)
    pltpu.sync_copy(iidx.at[pl.ds(base, b_per_w)], iidx_v)

    def job(j):
      if j < n_ch:
        return uidx_v, ut, uout, j * CH
      return iidx_v, it, iout, (j - n_ch) * CH

    gd = [None] * NBUF
    sd = [None] * NBUF
    for k in range(NBUF):
      idx_v, tab, _, off = job(k)
      gd[k] = pltpu.make_async_copy(tab.at[idx_v.at[pl.ds(off, CH)]],
                                    bufs[k], gsem[k])
      gd[k].start()
    for j in range(n_jobs):
      b = j % NBUF
      _, _, out, off = job(j)
      gd[b].wait()
      sd[b] = pltpu.make_async_copy(bufs[b], out.at[pl.ds(base + off, CH)],
                                    ssem[b])
      sd[b].start()
      nj = j + NBUF
      if nj < n_jobs:
        sd[b].wait()
        idx_v, tab, _, noff = job(nj)
        gd[b] = pltpu.make_async_copy(tab.at[idx_v.at[pl.ds(noff, CH)]],
                                      bufs[b], gsem[b])
        gd[b].start()
    for b in range(NBUF):
      sd[b].wait()

  return gather


def _mlp_body(u_ref, i_ref, a_ref, g_ref, c_ref, p_ref,
              at_ref, gt_ref, ct_ref,
              w1_ref, b1r,
              w2, b2r, w3, b3r, w4, b4r, o_ref):
  f32 = jnp.float32
  T = u_ref.shape[0]

  def onehot_emb(idx_ref, tab_ref):
    n = tab_ref.shape[0]
    iot = lax.broadcasted_iota(jnp.int32, (T, n), 1)
    oh = (iot == idx_ref[...]).astype(f32)
    return jnp.dot(oh, tab_ref[...], preferred_element_type=f32)

  w1 = w1_ref[...]
  h = jnp.dot(u_ref[...], w1[0:128], preferred_element_type=f32)
  h = h + jnp.dot(i_ref[...], w1[128:256], preferred_element_type=f32)
  h = h + jnp.dot(onehot_emb(a_ref, at_ref), w1[256:320],
                  preferred_element_type=f32)
  h = h + jnp.dot(onehot_emb(g_ref, gt_ref), w1[320:352],
                  preferred_element_type=f32)
  h = h + jnp.dot(onehot_emb(c_ref, ct_ref), w1[352:416],
                  preferred_element_type=f32)
  h = h + (p_ref[...] * (1.0 / 1000.0)) * w1[416:417]
  h = jnp.maximum(h + b1r[...], 0.0)
  h = jnp.maximum(jnp.dot(h, w2[...], preferred_element_type=f32) + b2r[...],
                  0.0)
  h = jnp.maximum(jnp.dot(h, w3[...], preferred_element_type=f32) + b3r[...],
                  0.0)
  o_ref[...] = jnp.dot(h, w4[...], preferred_element_type=f32) + b4r[...]


def kernel(user_id, age, gender, item_id, category, price,
           user_table, item_table, age_table, gender_table, cat_table,
           W1, b1, W2, b2, W3, b3, W4, b4):
  B = user_id.shape[0]

  uidx = user_id.reshape(B).astype(jnp.int32)
  iidx = item_id.reshape(B).astype(jnp.int32)

  u_emb, i_emb = _make_sc_gather(B)(uidx, iidx, user_table, item_table)

  TILE = 2048
  grid = (B // TILE,)

  def tile_spec(w):
    return pl.BlockSpec((TILE, w), lambda i: (i, 0))

  def full_spec(a):
    r, c = a.shape
    return pl.BlockSpec((r, c), lambda i: (0, 0))

  b1r = b1.reshape(1, -1)
  b2r = b2.reshape(1, -1)
  b3r = b3.reshape(1, -1)
  b4r = b4.reshape(1, -1)

  out = pl.pallas_call(
      _mlp_body,
      grid=grid,
      in_specs=[
          tile_spec(D), tile_spec(D),
          tile_spec(1), tile_spec(1), tile_spec(1), tile_spec(1),
          full_spec(age_table), full_spec(gender_table), full_spec(cat_table),
          full_spec(W1), full_spec(b1r),
          full_spec(W2), full_spec(b2r),
          full_spec(W3), full_spec(b3r),
          full_spec(W4), full_spec(b4r),
      ],
      out_specs=pl.BlockSpec((TILE, 1), lambda i: (i, 0)),
      out_shape=jax.ShapeDtypeStruct((B, 1), jnp.float32),
      compiler_params=pltpu.CompilerParams(
          dimension_semantics=("parallel",)),
  )(u_emb, i_emb, age, gender, category, price,
    age_table, gender_table, cat_table,
    W1, b1r, W2, b2r, W3, b3r, W4, b4r)

  return out[:, 0]


# speedup vs baseline: 1.2797x; 1.2681x over previous
import functools

import jax
import jax.numpy as jnp
from jax import lax
from jax.experimental import pallas as pl
from jax.experimental.pallas import tpu as pltpu
from jax.experimental.pallas import tpu_sc as plsc

D = 128


def _make_sc_gather(B):
  info = plsc.get_sparse_core_info()
  NC, NS = info.num_cores, info.num_subcores
  NW = NC * NS
  assert B % (8 * NW) == 0
  b_per_w = B // NW
  CH = 128
  n_ch = b_per_w // CH
  NBUF = 4
  n_jobs = 2 * n_ch
  mesh = plsc.VectorSubcoreMesh(core_axis_name="c", subcore_axis_name="s")

  out_type = (
      jax.ShapeDtypeStruct((B, D), jnp.float32),
      jax.ShapeDtypeStruct((B, D), jnp.float32),
  )

  @functools.partial(
      pl.kernel,
      mesh=mesh,
      out_type=out_type,
      scratch_types=(
          [pltpu.VMEM((b_per_w,), jnp.int32) for _ in range(2)]
          + [pltpu.VMEM((CH, D), jnp.float32) for _ in range(NBUF)]
          + [pltpu.SemaphoreType.DMA for _ in range(2 * NBUF)]
      ),
      compiler_params=pltpu.CompilerParams(use_tc_tiling_on_sc=True),
  )
  def gather(uidx, iidx, ut, it, uout, iout,
             uidx_v, iidx_v, b0, b1, b2, b3, *sems):
    bufs = (b0, b1, b2, b3)
    gsem = sems[:NBUF]
    ssem = sems[NBUF:]
    wid = lax.axis_index("s") * NC + lax.axis_index("c")
    base = wid * b_per_w
    pltpu.sync_copy(uidx.at[pl.ds(base, b_per_w)], uidx_v)
    pltpu.sync_copy(iidx.at[pl.ds(base, b_per_w)], iidx_v)

    def job(j):
      if j < n_ch:
        return uidx_v, ut, uout, j * CH
      return iidx_v, it, iout, (j - n_ch) * CH

    gd = [None] * NBUF
    sd = [None] * NBUF
    for k in range(NBUF):
      idx_v, tab, _, off = job(k)
      gd[k] = pltpu.make_async_copy(tab.at[idx_v.at[pl.ds(off, CH)]],
                                    bufs[k], gsem[k])
      gd[k].start()
    for j in range(n_jobs):
      b = j % NBUF
      _, _, out, off = job(j)
      gd[b].wait()
      sd[b] = pltpu.make_async_copy(bufs[b], out.at[pl.ds(base + off, CH)],
                                    ssem[b])
      sd[b].start()
      nj = j + NBUF
      if nj < n_jobs:
        sd[b].wait()
        idx_v, tab, _, noff = job(nj)
        gd[b] = pltpu.make_async_copy(tab.at[idx_v.at[pl.ds(noff, CH)]],
                                      bufs[b], gsem[b])
        gd[b].start()
    for b in range(NBUF):
      sd[b].wait()

  return gather


_DN_T = (((0,), (0,)), ((), ()))


def _mlp_body(u_ref, i_ref, a_ref, g_ref, c_ref, p_ref,
              at_ref, gt_ref, ct_ref,
              w1_ref, b1r,
              w2, b2r, w3, b3r, w4, b4r, o_ref):
  f32 = jnp.float32
  T = u_ref.shape[0]

  def onehot_emb(idx_ref, tab_ref):
    n = tab_ref.shape[0]
    iot = lax.broadcasted_iota(jnp.int32, (n, T), 0)
    oh = (iot == idx_ref[0]).astype(f32)
    return lax.dot_general(oh, tab_ref[...], _DN_T,
                           preferred_element_type=f32)

  w1 = w1_ref[...]
  h = jnp.dot(u_ref[...], w1[0:128], preferred_element_type=f32)
  h = h + jnp.dot(i_ref[...], w1[128:256], preferred_element_type=f32)
  h = h + jnp.dot(onehot_emb(a_ref, at_ref), w1[256:320],
                  preferred_element_type=f32)
  h = h + jnp.dot(onehot_emb(g_ref, gt_ref), w1[320:352],
                  preferred_element_type=f32)
  h = h + jnp.dot(onehot_emb(c_ref, ct_ref), w1[352:416],
                  preferred_element_type=f32)
  h = h + lax.dot_general(p_ref[0] * (1.0 / 1000.0), w1[416:417], _DN_T,
                          preferred_element_type=f32)
  h = jnp.maximum(h + b1r[...], 0.0)
  h = jnp.maximum(jnp.dot(h, w2[...], preferred_element_type=f32) + b2r[...],
                  0.0)
  h = jnp.maximum(jnp.dot(h, w3[...], preferred_element_type=f32) + b3r[...],
                  0.0)
  lo = lax.dot_general(w4[...], h, (((0,), (1,)), ((), ())),
                       preferred_element_type=f32) + b4r[...]
  o_ref[...] = lo.reshape(1, 1, T)


def kernel(user_id, age, gender, item_id, category, price,
           user_table, item_table, age_table, gender_table, cat_table,
           W1, b1, W2, b2, W3, b3, W4, b4):
  B = user_id.shape[0]

  uidx = user_id.reshape(B).astype(jnp.int32)
  iidx = item_id.reshape(B).astype(jnp.int32)

  u_emb, i_emb = _make_sc_gather(B)(uidx, iidx, user_table, item_table)

  TILE = 2048
  G = B // TILE
  grid = (G,)

  def tile_spec(w):
    return pl.BlockSpec((TILE, w), lambda i: (i, 0))

  def lane_spec():
    return pl.BlockSpec((1, 1, TILE), lambda i: (i, 0, 0))

  def full_spec(a):
    r, c = a.shape
    return pl.BlockSpec((r, c), lambda i: (0, 0))

  def lanes(x):
    return x.reshape(G, 1, TILE)

  b1r = b1.reshape(1, -1)
  b2r = b2.reshape(1, -1)
  b3r = b3.reshape(1, -1)
  b4r = b4.reshape(1, -1)

  out = pl.pallas_call(
      _mlp_body,
      grid=grid,
      in_specs=[
          tile_spec(D), tile_spec(D),
          lane_spec(), lane_spec(), lane_spec(), lane_spec(),
          full_spec(age_table), full_spec(gender_table), full_spec(cat_table),
          full_spec(W1), full_spec(b1r),
          full_spec(W2), full_spec(b2r),
          full_spec(W3), full_spec(b3r),
          full_spec(W4), full_spec(b4r),
      ],
      out_specs=pl.BlockSpec((1, 1, TILE), lambda i: (i, 0, 0)),
      out_shape=jax.ShapeDtypeStruct((G, 1, TILE), jnp.float32),
      compiler_params=pltpu.CompilerParams(
          dimension_semantics=("parallel",)),
  )(u_emb, i_emb, lanes(age), lanes(gender), lanes(category), lanes(price),
    age_table, gender_table, cat_table,
    W1, b1r, W2, b2r, W3, b3r, W4, b4r)

  return out.reshape(B)


# speedup vs baseline: 1.3174x; 1.0294x over previous
import functools

import jax
import jax.numpy as jnp
from jax import lax
from jax.experimental import pallas as pl
from jax.experimental.pallas import tpu as pltpu
from jax.experimental.pallas import tpu_sc as plsc

D = 128


def _make_sc_gather(B):
  info = plsc.get_sparse_core_info()
  NC, NS = info.num_cores, info.num_subcores
  NW = NC * NS
  assert B % (8 * NW) == 0
  b_per_w = B // NW
  CH = 128
  n_ch = b_per_w // CH
  NBUF = 4
  n_jobs = 2 * n_ch
  mesh = plsc.VectorSubcoreMesh(core_axis_name="c", subcore_axis_name="s")

  out_type = (
      jax.ShapeDtypeStruct((B, D), jnp.float32),
      jax.ShapeDtypeStruct((B, D), jnp.float32),
  )

  @functools.partial(
      pl.kernel,
      mesh=mesh,
      out_type=out_type,
      scratch_types=(
          [pltpu.VMEM((b_per_w,), jnp.int32) for _ in range(2)]
          + [pltpu.VMEM((CH, D), jnp.float32) for _ in range(NBUF)]
          + [pltpu.SemaphoreType.DMA for _ in range(2 * NBUF)]
      ),
      compiler_params=pltpu.CompilerParams(use_tc_tiling_on_sc=True),
  )
  def gather(uidx, iidx, ut, it, uout, iout,
             uidx_v, iidx_v, b0, b1, b2, b3, *sems):
    bufs = (b0, b1, b2, b3)
    gsem = sems[:NBUF]
    ssem = sems[NBUF:]
    wid = lax.axis_index("s") * NC + lax.axis_index("c")
    base = wid * b_per_w
    pltpu.sync_copy(uidx.at[pl.ds(base, b_per_w)], uidx_v)
    pltpu.sync_copy(iidx.at[pl.ds(base, b_per_w)], iidx_v)

    def job(j):
      if j < n_ch:
        return uidx_v, ut, uout, j * CH
      return iidx_v, it, iout, (j - n_ch) * CH

    gd = [None] * NBUF
    sd = [None] * NBUF
    for k in range(NBUF):
      idx_v, tab, _, off = job(k)
      gd[k] = pltpu.make_async_copy(tab.at[idx_v.at[pl.ds(off, CH)]],
                                    bufs[k], gsem[k])
      gd[k].start()
    for j in range(n_jobs):
      b = j % NBUF
      _, _, out, off = job(j)
      gd[b].wait()
      sd[b] = pltpu.make_async_copy(bufs[b], out.at[pl.ds(base + off, CH)],
                                    ssem[b])
      sd[b].start()
      nj = j + NBUF
      if nj < n_jobs:
        sd[b].wait()
        idx_v, tab, _, noff = job(nj)
        gd[b] = pltpu.make_async_copy(tab.at[idx_v.at[pl.ds(noff, CH)]],
                                      bufs[b], gsem[b])
        gd[b].start()
    for b in range(NBUF):
      sd[b].wait()

  return gather


_DN_T = (((0,), (0,)), ((), ()))


def _mlp_body(u_ref, i_ref, a_ref, g_ref, c_ref, p_ref,
              at_ref, gt_ref, ct_ref,
              w1_ref, b1r,
              w2, b2r, w3, b3r, w4, b4r, o_ref):
  f32 = jnp.float32
  T = u_ref.shape[0]

  bf16 = jnp.bfloat16

  def onehot_emb(idx_ref, tab_ref):
    n = tab_ref.shape[0]
    iot = lax.broadcasted_iota(jnp.int32, (n, T), 0)
    oh = (iot == idx_ref[0]).astype(bf16)
    return lax.dot_general(oh, tab_ref[...], _DN_T,
                           preferred_element_type=f32)

  w1 = w1_ref[...]
  h = jnp.dot(u_ref[...].astype(bf16), w1[0:128], preferred_element_type=f32)
  h = h + jnp.dot(i_ref[...].astype(bf16), w1[128:256],
                  preferred_element_type=f32)
  h = h + jnp.dot(onehot_emb(a_ref, at_ref).astype(bf16), w1[256:320],
                  preferred_element_type=f32)
  h = h + jnp.dot(onehot_emb(g_ref, gt_ref).astype(bf16), w1[320:352],
                  preferred_element_type=f32)
  h = h + jnp.dot(onehot_emb(c_ref, ct_ref).astype(bf16), w1[352:416],
                  preferred_element_type=f32)
  h = h + lax.dot_general(p_ref[0] * (1.0 / 1000.0), w1[416:417].astype(f32),
                          _DN_T, preferred_element_type=f32)
  h = jnp.maximum(h + b1r[...], 0.0)
  h = jnp.maximum(
      jnp.dot(h.astype(bf16), w2[...], preferred_element_type=f32) + b2r[...],
      0.0)
  h = jnp.maximum(
      jnp.dot(h.astype(bf16), w3[...], preferred_element_type=f32) + b3r[...],
      0.0)
  lo = lax.dot_general(w4[...], h.astype(bf16), (((0,), (1,)), ((), ())),
                       preferred_element_type=f32) + b4r[...]
  o_ref[...] = lo.reshape(1, 1, T)


def kernel(user_id, age, gender, item_id, category, price,
           user_table, item_table, age_table, gender_table, cat_table,
           W1, b1, W2, b2, W3, b3, W4, b4):
  B = user_id.shape[0]

  uidx = user_id.reshape(B).astype(jnp.int32)
  iidx = item_id.reshape(B).astype(jnp.int32)

  u_emb, i_emb = _make_sc_gather(B)(uidx, iidx, user_table, item_table)

  TILE = 2048
  G = B // TILE
  grid = (G,)

  def tile_spec(w):
    return pl.BlockSpec((TILE, w), lambda i: (i, 0))

  def lane_spec():
    return pl.BlockSpec((1, 1, TILE), lambda i: (i, 0, 0))

  def full_spec(a):
    r, c = a.shape
    return pl.BlockSpec((r, c), lambda i: (0, 0))

  def lanes(x):
    return x.reshape(G, 1, TILE)

  b1r = b1.reshape(1, -1)
  b2r = b2.reshape(1, -1)
  b3r = b3.reshape(1, -1)
  b4r = b4.reshape(1, -1)

  bf16 = jnp.bfloat16
  at_b = age_table.astype(bf16)
  gt_b = gender_table.astype(bf16)
  ct_b = cat_table.astype(bf16)
  w1_b = W1.astype(bf16)
  w2_b = W2.astype(bf16)
  w3_b = W3.astype(bf16)
  w4_b = W4.astype(bf16)

  out = pl.pallas_call(
      _mlp_body,
      grid=grid,
      in_specs=[
          tile_spec(D), tile_spec(D),
          lane_spec(), lane_spec(), lane_spec(), lane_spec(),
          full_spec(at_b), full_spec(gt_b), full_spec(ct_b),
          full_spec(w1_b), full_spec(b1r),
          full_spec(w2_b), full_spec(b2r),
          full_spec(w3_b), full_spec(b3r),
          full_spec(w4_b), full_spec(b4r),
      ],
      out_specs=pl.BlockSpec((1, 1, TILE), lambda i: (i, 0, 0)),
      out_shape=jax.ShapeDtypeStruct((G, 1, TILE), jnp.float32),
      compiler_params=pltpu.CompilerParams(
          dimension_semantics=("parallel",)),
  )(u_emb, i_emb, lanes(age), lanes(gender), lanes(category), lanes(price),
    at_b, gt_b, ct_b,
    w1_b, b1r, w2_b, b2r, w3_b, b3r, w4_b, b4r)

  return out.reshape(B)
